# Initial kernel scaffold; baseline (speedup 1.0000x reference)
import jax
import jax.numpy as jnp
from jax.experimental import pallas as pl


def kernel(h, x, vel_all, edge_index, tc_h_wr, tc_h_wi, tc_v_wr, tc_v_wi, W_e1, b_e1, W_e2, b_e2, W_g, b_g, W_n1, b_n1, W_n2, b_n2):
    raise NotImplementedError("write your pallas kernel here")


# speedup vs baseline: 7.8483x; 7.8483x over previous
import functools

import jax
import jax.numpy as jnp
import numpy as np
from jax import lax
from jax.experimental import pallas as pl
from jax.experimental.pallas import tpu as pltpu
from jax.experimental.pallas import tpu_sc as plsc

_CR = np.array(
    [[[0.25, 0.25, 0.25, 0.25]] * 4,
     [[0.5, 0.0, -0.5, 0.0],
      [0.0, 0.5, 0.0, -0.5],
      [-0.5, 0.0, 0.5, 0.0],
      [0.0, -0.5, 0.0, 0.5]],
     [[0.25, -0.25, 0.25, -0.25],
      [-0.25, 0.25, -0.25, 0.25],
      [0.25, -0.25, 0.25, -0.25],
      [-0.25, 0.25, -0.25, 0.25]]], dtype=np.float32)
_CI = np.array(
    [[[0.0] * 4] * 4,
     [[0.0, -0.5, 0.0, 0.5],
      [0.5, 0.0, -0.5, 0.0],
      [0.0, 0.5, 0.0, -0.5],
      [-0.5, 0.0, 0.5, 0.0]],
     [[0.0] * 4] * 4], dtype=np.float32)

_NC, _NS = 2, 16


def _silu(v):
    return v * jax.nn.sigmoid(v)


def _prep_body(T, NB, C, h_ref, vel_ref, m2_ref, w1s_ref, w1d_ref, mv_ref,
               h2_ref, a_ref, b_ref, vout_ref):
    for u in range(T):
        acc = jnp.zeros((NB, C), jnp.float32)
        for t in range(T):
            acc = acc + jnp.dot(h_ref[t], m2_ref[t, u],
                                preferred_element_type=jnp.float32)
        h2 = h_ref[u] + jnp.where(acc > 0, acc, 0.2 * acc)
        h2_ref[u] = h2
        a_ref[u] = jnp.dot(h2, w1s_ref[...], preferred_element_type=jnp.float32)
        b_ref[u] = jnp.dot(h2, w1d_ref[...], preferred_element_type=jnp.float32)
    v = vel_ref[...]
    vout_ref[...] = v + jnp.dot(v, mv_ref[...], preferred_element_type=jnp.float32)


def _prep(h, vel12, m2, w1s, w1d, mv12, NB):
    T, BN, C = h.shape
    grid = BN // NB
    f32 = jnp.float32
    return pl.pallas_call(
        functools.partial(_prep_body, T, NB, C),
        grid=(grid,),
        in_specs=[
            pl.BlockSpec((T, NB, C), lambda i: (0, i, 0)),
            pl.BlockSpec((NB, 12), lambda i: (i, 0)),
            pl.BlockSpec((T, T, C, C), lambda i: (0, 0, 0, 0)),
            pl.BlockSpec((C, C), lambda i: (0, 0)),
            pl.BlockSpec((C, C), lambda i: (0, 0)),
            pl.BlockSpec((12, 12), lambda i: (0, 0)),
        ],
        out_specs=[
            pl.BlockSpec((T, NB, C), lambda i: (0, i, 0)),
            pl.BlockSpec((T, NB, C), lambda i: (0, i, 0)),
            pl.BlockSpec((T, NB, C), lambda i: (0, i, 0)),
            pl.BlockSpec((NB, 12), lambda i: (i, 0)),
        ],
        out_shape=[
            jax.ShapeDtypeStruct((T, BN, C), f32),
            jax.ShapeDtypeStruct((T, BN, C), f32),
            jax.ShapeDtypeStruct((T, BN, C), f32),
            jax.ShapeDtypeStruct((BN, 12), f32),
        ],
    )(h, vel12, m2, w1s, w1d, mv12)


def _sc_gather(table, idx, P):
    V, D = table.shape
    B = idx.shape[0]
    NW = _NC * _NS
    per_w = B // NW
    pieces = per_w // P
    mesh = plsc.VectorSubcoreMesh(core_axis_name="c", subcore_axis_name="s")

    @functools.partial(
        pl.kernel,
        out_type=jax.ShapeDtypeStruct((B, D), table.dtype),
        mesh=mesh,
        scratch_types=[
            pltpu.VMEM((P,), jnp.int32),
            pltpu.VMEM((P, D), table.dtype),
            pltpu.SemaphoreType.DMA,
        ],
    )
    def k(table_hbm, idx_hbm, out_hbm, idx_v, rows_v, sem):
        wid = lax.axis_index("s") * _NC + lax.axis_index("c")
        base = wid * per_w

        def body(j, carry):
            off = base + j * P
            pltpu.sync_copy(idx_hbm.at[pl.ds(off, P)], idx_v)
            pltpu.async_copy(table_hbm.at[idx_v], rows_v, sem).wait()
            pltpu.sync_copy(rows_v, out_hbm.at[pl.ds(off, P)])
            return carry

        lax.fori_loop(0, pieces, body, 0)

    return k(table, idx)


def _sc_d2(xflat, src, dst):
    E = src.shape[0]
    NW = _NC * _NS
    per_w = E // NW
    mesh = plsc.VectorSubcoreMesh(core_axis_name="c", subcore_axis_name="s")

    @functools.partial(
        pl.kernel,
        out_type=jax.ShapeDtypeStruct((E,), jnp.float32),
        mesh=mesh,
        compiler_params=pltpu.CompilerParams(needs_layout_passes=False),
        scratch_types=[
            pltpu.VMEM(xflat.shape, jnp.float32),
            pltpu.VMEM((per_w,), jnp.int32),
            pltpu.VMEM((per_w,), jnp.int32),
            pltpu.VMEM((per_w,), jnp.float32),
        ],
    )
    def k(x_hbm, src_hbm, dst_hbm, out_hbm, x_v, src_v, dst_v, d2_v):
        wid = lax.axis_index("s") * _NC + lax.axis_index("c")
        base = wid * per_w
        pltpu.sync_copy(x_hbm, x_v)
        pltpu.sync_copy(src_hbm.at[pl.ds(base, per_w)], src_v)
        pltpu.sync_copy(dst_hbm.at[pl.ds(base, per_w)], dst_v)

        def body(j, carry):
            si = src_v[pl.ds(j * 16, 16)] * 4
            di = dst_v[pl.ds(j * 16, 16)] * 4
            acc = jnp.zeros((16,), jnp.float32)
            for c in range(3):
                a = plsc.load_gather(x_v, [si + c])
                b = plsc.load_gather(x_v, [di + c])
                acc = acc + (a - b) * (a - b)
            d2_v[pl.ds(j * 16, 16)] = acc
            return carry

        lax.fori_loop(0, per_w // 16, body, 0)
        pltpu.sync_copy(d2_v, out_hbm.at[pl.ds(base, per_w)])

    return k(xflat, src, dst)


def _edge_body(EB, ga_ref, gb_ref, d2_ref, we2_ref, consts_ref, out_ref):
    d2 = d2_ref[...]
    w_d = consts_ref[0:1, :]
    b_e1 = consts_ref[1:2, :]
    b_e2 = consts_ref[2:3, :]
    w_g = consts_ref[3:4, :]
    b_g = consts_ref[4:5, 0:1]
    pre = ga_ref[...] + gb_ref[...] + d2 * w_d + b_e1
    m = _silu(pre)
    m = _silu(jnp.dot(m, we2_ref[...], preferred_element_type=jnp.float32) + b_e2)
    g = jnp.sum(m * w_g, axis=1, keepdims=True) + b_g
    out_ref[...] = m * jax.nn.sigmoid(g)


def _edge_mlp(ga, gb, d2, we2, consts, NEB):
    TE, C = ga.shape
    E = d2.shape[0]
    EB = E // NEB
    grid = TE // NEB
    return pl.pallas_call(
        functools.partial(_edge_body, EB),
        grid=(grid,),
        in_specs=[
            pl.BlockSpec((NEB, C), lambda i: (i, 0)),
            pl.BlockSpec((NEB, C), lambda i: (i, 0)),
            pl.BlockSpec((NEB, 1), lambda i: (i % EB, 0)),
            pl.BlockSpec((C, C), lambda i: (0, 0)),
            pl.BlockSpec((8, C), lambda i: (0, 0)),
        ],
        out_specs=pl.BlockSpec((NEB, C), lambda i: (i, 0)),
        out_shape=jax.ShapeDtypeStruct((TE, C), jnp.float32),
    )(ga, gb, d2.reshape(E, 1), we2, consts)


def _sc_scatter(m_all, dst, zeros_nc, P):
    T, E, C = m_all.shape
    BNP = zeros_nc.shape[0]
    per_tile = E // (_NC * _NS)
    pieces = per_tile // P
    rows_per_tile = BNP // _NS
    mesh = plsc.VectorSubcoreMesh(core_axis_name="c", subcore_axis_name="s")

    @functools.partial(
        pl.kernel,
        out_type=jax.ShapeDtypeStruct((_NC, T, BNP, C), jnp.float32),
        mesh=mesh,
        scratch_types=[
            pltpu.VMEM((P,), jnp.int32),
            pltpu.VMEM((P, C), jnp.float32),
            pltpu.VMEM_SHARED((BNP, C), jnp.float32),
            pltpu.SemaphoreType.DMA,
        ],
    )
    def k(m_hbm, dst_hbm, zero_hbm, out_hbm, idx_v, rows_v, agg_sh, sem):
        c = lax.axis_index("c")
        s = lax.axis_index("s")
        tile_base = (c * _NS + s) * per_tile
        row0 = s * rows_per_tile
        for t in range(T):
            pltpu.sync_copy(zero_hbm.at[pl.ds(row0, rows_per_tile)],
                            agg_sh.at[pl.ds(row0, rows_per_tile)])
            plsc.subcore_barrier()

            def body(j, carry):
                off = tile_base + j * P
                pltpu.sync_copy(dst_hbm.at[pl.ds(off, P)], idx_v)
                pltpu.async_copy(m_hbm.at[t, pl.ds(off, P)], rows_v, sem).wait()
                pltpu.sync_copy(rows_v, agg_sh.at[idx_v], add=True)
                return carry

            lax.fori_loop(0, pieces, body, 0)
            plsc.subcore_barrier()
            pltpu.sync_copy(agg_sh.at[pl.ds(row0, rows_per_tile)],
                            out_hbm.at[c, t, pl.ds(row0, rows_per_tile)])
            plsc.subcore_barrier()

    return k(m_all, dst, zeros_nc)


def _node_body(n1a_ref, n1b_ref, n2_ref, consts_ref, h2_ref, agg_ref, out_ref):
    b_n1 = consts_ref[0:1, :]
    b_n2 = consts_ref[1:2, :]
    h2 = h2_ref[...]
    agg = agg_ref[0, 0] + agg_ref[1, 0]
    acc = (jnp.dot(h2, n1a_ref[...], preferred_element_type=jnp.float32)
           + jnp.dot(agg, n1b_ref[...], preferred_element_type=jnp.float32)
           + b_n1)
    u = _silu(acc)
    out_ref[...] = h2 + jnp.dot(u, n2_ref[...],
                                preferred_element_type=jnp.float32) + b_n2


def _node_mlp(h2r, agg4, n1a, n1b, n2, consts, NB):
    TB, C = h2r.shape
    T = agg4.shape[1]
    BN = TB // T
    bpt = BN // NB
    grid = TB // NB
    return pl.pallas_call(
        _node_body,
        grid=(grid,),
        in_specs=[
            pl.BlockSpec((C, C), lambda i: (0, 0)),
            pl.BlockSpec((C, C), lambda i: (0, 0)),
            pl.BlockSpec((C, C), lambda i: (0, 0)),
            pl.BlockSpec((8, C), lambda i: (0, 0)),
            pl.BlockSpec((NB, C), lambda i: (i, 0)),
            pl.BlockSpec((2, 1, NB, C), lambda i: (0, i // bpt, i % bpt, 0)),
        ],
        out_specs=pl.BlockSpec((NB, C), lambda i: (i, 0)),
        out_shape=jax.ShapeDtypeStruct((TB, C), jnp.float32),
    )(n1a, n1b, n2, consts, h2r, agg4)


def kernel(h, x, vel_all, edge_index, tc_h_wr, tc_h_wi, tc_v_wr, tc_v_wi,
           W_e1, b_e1, W_e2, b_e2, W_g, b_g, W_n1, b_n1, W_n2, b_n2):
    T, BN, C = h.shape
    E = edge_index.shape[1]
    f32 = jnp.float32
    cr = jnp.asarray(_CR)
    ci = jnp.asarray(_CI)

    m2 = (jnp.einsum('iok,ktu->tuio', tc_h_wr, cr)
          + jnp.einsum('iok,ktu->tuio', tc_h_wi, ci))
    mv = (jnp.einsum('k,ktu->tu', tc_v_wr[0, 0], cr)
          + jnp.einsum('k,ktu->tu', tc_v_wi[0, 0], ci))
    mv12 = jnp.kron(mv, jnp.eye(3, dtype=f32))
    w1s = W_e1[:C]
    w1d = W_e1[C:2 * C]
    w_d = W_e1[2 * C]
    n1a = W_n1[:C]
    n1b = W_n1[C:]

    vel12 = vel_all.reshape(BN, T * 3)
    h2, a_t, b_t, vout = _prep(h, vel12, m2, w1s, w1d, mv12, NB=1000)
    vel_new = vout.reshape(BN, T, 3)

    src = edge_index[0].astype(jnp.int32)
    dst = edge_index[1].astype(jnp.int32)
    toff = (jnp.arange(T, dtype=jnp.int32) * BN)[:, None]
    src_all = (src[None, :] + toff).reshape(-1)
    dst_all = (dst[None, :] + toff).reshape(-1)

    xflat = jnp.zeros((BN, 4), f32).at[:, :3].set(x).reshape(BN * 4)
    d2 = _sc_d2(xflat, src, dst)
    ga = _sc_gather(a_t.reshape(T * BN, C), src_all, P=800)
    gb = _sc_gather(b_t.reshape(T * BN, C), dst_all, P=800)

    consts_e = (jnp.zeros((8, C), f32)
                .at[0].set(w_d).at[1].set(b_e1).at[2].set(b_e2)
                .at[3].set(W_g[:, 0]).at[4, 0].set(b_g[0]))
    m_edges = _edge_mlp(ga, gb, d2, W_e2, consts_e, NEB=2000)

    BNP = ((BN // _NS + 7) // 8 * 8) * _NS
    zeros_nc = jnp.zeros((BNP, C), f32)
    agg = _sc_scatter(m_edges.reshape(T, E, C), dst, zeros_nc, P=200)

    consts_n = jnp.zeros((8, C), f32).at[0].set(b_n1).at[1].set(b_n2)
    h_out = _node_mlp(h2.reshape(T * BN, C), agg,
                      n1a, n1b, W_n2, consts_n, NB=2000)
    return h_out.reshape(T, BN, C), vel_new


# speedup vs baseline: 9.9472x; 1.2674x over previous
import functools

import jax
import jax.numpy as jnp
import numpy as np
from jax import lax
from jax.experimental import pallas as pl
from jax.experimental.pallas import tpu as pltpu
from jax.experimental.pallas import tpu_sc as plsc

_CR = np.array(
    [[[0.25, 0.25, 0.25, 0.25]] * 4,
     [[0.5, 0.0, -0.5, 0.0],
      [0.0, 0.5, 0.0, -0.5],
      [-0.5, 0.0, 0.5, 0.0],
      [0.0, -0.5, 0.0, 0.5]],
     [[0.25, -0.25, 0.25, -0.25],
      [-0.25, 0.25, -0.25, 0.25],
      [0.25, -0.25, 0.25, -0.25],
      [-0.25, 0.25, -0.25, 0.25]]], dtype=np.float32)
_CI = np.array(
    [[[0.0] * 4] * 4,
     [[0.0, -0.5, 0.0, 0.5],
      [0.5, 0.0, -0.5, 0.0],
      [0.0, 0.5, 0.0, -0.5],
      [-0.5, 0.0, 0.5, 0.0]],
     [[0.0] * 4] * 4], dtype=np.float32)

_NC, _NS = 2, 16


def _silu(v):
    return v * jax.nn.sigmoid(v)


def _pack_bf16_pairs(a):
    half = a.shape[1] // 2
    ai = jax.lax.bitcast_convert_type(
        a.astype(jnp.bfloat16).astype(jnp.float32), jnp.uint32)
    lo = jax.lax.shift_right_logical(ai[:, :half], jnp.uint32(16))
    hi = jnp.bitwise_and(ai[:, half:], jnp.uint32(0xFFFF0000))
    return jax.lax.bitcast_convert_type(jnp.bitwise_or(lo, hi), jnp.float32)


def _unpack_bf16_pairs(w):
    wi = jax.lax.bitcast_convert_type(w, jnp.uint32)
    lo = jax.lax.bitcast_convert_type(
        jax.lax.shift_left(wi, jnp.uint32(16)), jnp.float32)
    hi = jax.lax.bitcast_convert_type(
        jnp.bitwise_and(wi, jnp.uint32(0xFFFF0000)), jnp.float32)
    return jnp.concatenate([lo, hi], axis=1)


def _prep_body(T, NB, C, h_ref, vel_ref, m2_ref, w1s_ref, w1d_ref, mv_ref,
               h2_ref, a_ref, b_ref, vout_ref):
    pa, pb = [], []
    for u in range(T):
        acc = jnp.zeros((NB, C), jnp.float32)
        for t in range(T):
            acc = acc + jnp.dot(h_ref[t], m2_ref[t, u],
                                preferred_element_type=jnp.float32)
        h2 = h_ref[u] + jnp.where(acc > 0, acc, 0.2 * acc)
        h2_ref[u] = h2
        pa.append(_pack_bf16_pairs(
            jnp.dot(h2, w1s_ref[...], preferred_element_type=jnp.float32)))
        pb.append(_pack_bf16_pairs(
            jnp.dot(h2, w1d_ref[...], preferred_element_type=jnp.float32)))
    for p in range(T // 2):
        a_ref[p] = jnp.concatenate([pa[2 * p], pa[2 * p + 1]], axis=1)
        b_ref[p] = jnp.concatenate([pb[2 * p], pb[2 * p + 1]], axis=1)
    v = vel_ref[...]
    vout_ref[...] = v + jnp.dot(v, mv_ref[...], preferred_element_type=jnp.float32)


def _prep(h, vel12, m2, w1s, w1d, mv12, NB):
    T, BN, C = h.shape
    grid = BN // NB
    f32 = jnp.float32
    return pl.pallas_call(
        functools.partial(_prep_body, T, NB, C),
        grid=(grid,),
        in_specs=[
            pl.BlockSpec((T, NB, C), lambda i: (0, i, 0)),
            pl.BlockSpec((NB, 12), lambda i: (i, 0)),
            pl.BlockSpec((T, T, C, C), lambda i: (0, 0, 0, 0)),
            pl.BlockSpec((C, C), lambda i: (0, 0)),
            pl.BlockSpec((C, C), lambda i: (0, 0)),
            pl.BlockSpec((12, 12), lambda i: (0, 0)),
        ],
        out_specs=[
            pl.BlockSpec((T, NB, C), lambda i: (0, i, 0)),
            pl.BlockSpec((T // 2, NB, C), lambda i: (0, i, 0)),
            pl.BlockSpec((T // 2, NB, C), lambda i: (0, i, 0)),
            pl.BlockSpec((NB, 12), lambda i: (i, 0)),
        ],
        out_shape=[
            jax.ShapeDtypeStruct((T, BN, C), f32),
            jax.ShapeDtypeStruct((T // 2, BN, C), f32),
            jax.ShapeDtypeStruct((T // 2, BN, C), f32),
            jax.ShapeDtypeStruct((BN, 12), f32),
        ],
    )(h, vel12, m2, w1s, w1d, mv12)


def _sc_gather(table, idx, P):
    V, D = table.shape
    B = idx.shape[0]
    NW = _NC * _NS
    per_w = B // NW
    pieces = per_w // P
    mesh = plsc.VectorSubcoreMesh(core_axis_name="c", subcore_axis_name="s")

    @functools.partial(
        pl.kernel,
        out_type=jax.ShapeDtypeStruct((B, D), table.dtype),
        mesh=mesh,
        scratch_types=[
            pltpu.VMEM((P,), jnp.int32),
            pltpu.VMEM((P, D), table.dtype),
            pltpu.SemaphoreType.DMA,
        ],
    )
    def k(table_hbm, idx_hbm, out_hbm, idx_v, rows_v, sem):
        wid = lax.axis_index("s") * _NC + lax.axis_index("c")
        base = wid * per_w

        def body(j, carry):
            off = base + j * P
            pltpu.sync_copy(idx_hbm.at[pl.ds(off, P)], idx_v)
            pltpu.async_copy(table_hbm.at[idx_v], rows_v, sem).wait()
            pltpu.sync_copy(rows_v, out_hbm.at[pl.ds(off, P)])
            return carry

        lax.fori_loop(0, pieces, body, 0)

    return k(table, idx)


def _sc_d2(xflat, src, dst):
    E = src.shape[0]
    NW = _NC * _NS
    per_w = E // NW
    mesh = plsc.VectorSubcoreMesh(core_axis_name="c", subcore_axis_name="s")

    @functools.partial(
        pl.kernel,
        out_type=jax.ShapeDtypeStruct((E,), jnp.float32),
        mesh=mesh,
        compiler_params=pltpu.CompilerParams(needs_layout_passes=False),
        scratch_types=[
            pltpu.VMEM(xflat.shape, jnp.float32),
            pltpu.VMEM((per_w,), jnp.int32),
            pltpu.VMEM((per_w,), jnp.int32),
            pltpu.VMEM((per_w,), jnp.float32),
        ],
    )
    def k(x_hbm, src_hbm, dst_hbm, out_hbm, x_v, src_v, dst_v, d2_v):
        wid = lax.axis_index("s") * _NC + lax.axis_index("c")
        base = wid * per_w
        pltpu.sync_copy(x_hbm, x_v)
        pltpu.sync_copy(src_hbm.at[pl.ds(base, per_w)], src_v)
        pltpu.sync_copy(dst_hbm.at[pl.ds(base, per_w)], dst_v)

        def body(j, carry):
            si = src_v[pl.ds(j * 16, 16)] * 4
            di = dst_v[pl.ds(j * 16, 16)] * 4
            acc = jnp.zeros((16,), jnp.float32)
            for c in range(3):
                a = plsc.load_gather(x_v, [si + c])
                b = plsc.load_gather(x_v, [di + c])
                acc = acc + (a - b) * (a - b)
            d2_v[pl.ds(j * 16, 16)] = acc
            return carry

        lax.fori_loop(0, per_w // 16, body, 0)
        pltpu.sync_copy(d2_v, out_hbm.at[pl.ds(base, per_w)])

    return k(xflat, src, dst)


def _edge_body(EB, C, ga_ref, gb_ref, d2_ref, we2_ref, consts_ref, out_ref):
    d2 = d2_ref[...]
    w_d = consts_ref[0:1, :]
    b_e1 = consts_ref[1:2, :]
    b_e2 = consts_ref[2:3, :]
    w_g = consts_ref[3:4, :]
    b_g = consts_ref[4:5, 0:1]
    H = C // 2
    ga = ga_ref[...]
    gb = gb_ref[...]
    for parity in range(2):
        a = _unpack_bf16_pairs(ga[:, parity * H:(parity + 1) * H])
        b = _unpack_bf16_pairs(gb[:, parity * H:(parity + 1) * H])
        pre = a + b + d2 * w_d + b_e1
        m = _silu(pre)
        m = _silu(jnp.dot(m.astype(jnp.bfloat16), we2_ref[...],
                          preferred_element_type=jnp.float32) + b_e2)
        g = jnp.sum(m * w_g, axis=1, keepdims=True) + b_g
        out_ref[0, parity] = m * jax.nn.sigmoid(g)


def _edge_mlp(ga, gb, d2, we2, consts, NEB):
    TE2, C = ga.shape
    E = d2.shape[0]
    EB = E // NEB
    grid = TE2 // NEB
    return pl.pallas_call(
        functools.partial(_edge_body, EB, C),
        grid=(grid,),
        in_specs=[
            pl.BlockSpec((NEB, C), lambda i: (i, 0)),
            pl.BlockSpec((NEB, C), lambda i: (i, 0)),
            pl.BlockSpec((NEB, 1), lambda i: (i % EB, 0)),
            pl.BlockSpec((C, C), lambda i: (0, 0)),
            pl.BlockSpec((8, C), lambda i: (0, 0)),
        ],
        out_specs=pl.BlockSpec((1, 2, NEB, C), lambda i: (i // EB, 0, i % EB, 0)),
        out_shape=jax.ShapeDtypeStruct((2, 2, E, C), jnp.float32),
    )(ga, gb, d2.reshape(E, 1), we2.astype(jnp.bfloat16), consts)


def _sc_scatter(m_all, dst, zeros_nc, P):
    T, E, C = m_all.shape
    BNP = zeros_nc.shape[0]
    per_tile = E // (_NC * _NS)
    pieces = per_tile // P
    rows_per_tile = BNP // _NS
    mesh = plsc.VectorSubcoreMesh(core_axis_name="c", subcore_axis_name="s")

    @functools.partial(
        pl.kernel,
        out_type=jax.ShapeDtypeStruct((_NC, T, BNP, C), jnp.float32),
        mesh=mesh,
        scratch_types=[
            pltpu.VMEM((P,), jnp.int32),
            pltpu.VMEM((P, C), jnp.float32),
            pltpu.VMEM_SHARED((BNP, C), jnp.float32),
            pltpu.SemaphoreType.DMA,
        ],
    )
    def k(m_hbm, dst_hbm, zero_hbm, out_hbm, idx_v, rows_v, agg_sh, sem):
        c = lax.axis_index("c")
        s = lax.axis_index("s")
        tile_base = (c * _NS + s) * per_tile
        row0 = s * rows_per_tile
        for t in range(T):
            pltpu.sync_copy(zero_hbm.at[pl.ds(row0, rows_per_tile)],
                            agg_sh.at[pl.ds(row0, rows_per_tile)])
            plsc.subcore_barrier()

            def body(j, carry):
                off = tile_base + j * P
                pltpu.sync_copy(dst_hbm.at[pl.ds(off, P)], idx_v)
                pltpu.async_copy(m_hbm.at[t, pl.ds(off, P)], rows_v, sem).wait()
                pltpu.sync_copy(rows_v, agg_sh.at[idx_v], add=True)
                return carry

            lax.fori_loop(0, pieces, body, 0)
            plsc.subcore_barrier()
            pltpu.sync_copy(agg_sh.at[pl.ds(row0, rows_per_tile)],
                            out_hbm.at[c, t, pl.ds(row0, rows_per_tile)])
            plsc.subcore_barrier()

    return k(m_all, dst, zeros_nc)


def _node_body(n1a_ref, n1b_ref, n2_ref, consts_ref, h2_ref, agg_ref, out_ref):
    b_n1 = consts_ref[0:1, :]
    b_n2 = consts_ref[1:2, :]
    h2 = h2_ref[...]
    agg = agg_ref[0, 0] + agg_ref[1, 0]
    acc = (jnp.dot(h2, n1a_ref[...], preferred_element_type=jnp.float32)
           + jnp.dot(agg, n1b_ref[...], preferred_element_type=jnp.float32)
           + b_n1)
    u = _silu(acc)
    out_ref[...] = h2 + jnp.dot(u, n2_ref[...],
                                preferred_element_type=jnp.float32) + b_n2


def _node_mlp(h2r, agg4, n1a, n1b, n2, consts, NB):
    TB, C = h2r.shape
    T = agg4.shape[1]
    BN = TB // T
    bpt = BN // NB
    grid = TB // NB
    return pl.pallas_call(
        _node_body,
        grid=(grid,),
        in_specs=[
            pl.BlockSpec((C, C), lambda i: (0, 0)),
            pl.BlockSpec((C, C), lambda i: (0, 0)),
            pl.BlockSpec((C, C), lambda i: (0, 0)),
            pl.BlockSpec((8, C), lambda i: (0, 0)),
            pl.BlockSpec((NB, C), lambda i: (i, 0)),
            pl.BlockSpec((2, 1, NB, C), lambda i: (0, i // bpt, i % bpt, 0)),
        ],
        out_specs=pl.BlockSpec((NB, C), lambda i: (i, 0)),
        out_shape=jax.ShapeDtypeStruct((TB, C), jnp.float32),
    )(n1a, n1b, n2, consts, h2r, agg4)


def kernel(h, x, vel_all, edge_index, tc_h_wr, tc_h_wi, tc_v_wr, tc_v_wi,
           W_e1, b_e1, W_e2, b_e2, W_g, b_g, W_n1, b_n1, W_n2, b_n2):
    T, BN, C = h.shape
    E = edge_index.shape[1]
    f32 = jnp.float32
    cr = jnp.asarray(_CR)
    ci = jnp.asarray(_CI)

    m2 = (jnp.einsum('iok,ktu->tuio', tc_h_wr, cr)
          + jnp.einsum('iok,ktu->tuio', tc_h_wi, ci))
    mv = (jnp.einsum('k,ktu->tu', tc_v_wr[0, 0], cr)
          + jnp.einsum('k,ktu->tu', tc_v_wi[0, 0], ci))
    mv12 = jnp.kron(mv, jnp.eye(3, dtype=f32))
    w1s = W_e1[:C]
    w1d = W_e1[C:2 * C]
    w_d = W_e1[2 * C]
    n1a = W_n1[:C]
    n1b = W_n1[C:]

    vel12 = vel_all.reshape(BN, T * 3)
    h2, a_t, b_t, vout = _prep(h, vel12, m2, w1s, w1d, mv12, NB=1000)
    vel_new = vout.reshape(BN, T, 3)

    src = edge_index[0].astype(jnp.int32)
    dst = edge_index[1].astype(jnp.int32)
    toff = (jnp.arange(T // 2, dtype=jnp.int32) * BN)[:, None]
    src_all = (src[None, :] + toff).reshape(-1)
    dst_all = (dst[None, :] + toff).reshape(-1)

    xflat = jnp.zeros((BN, 4), f32).at[:, :3].set(x).reshape(BN * 4)
    d2 = _sc_d2(xflat, src, dst)
    ga = _sc_gather(a_t.reshape(T // 2 * BN, C), src_all, P=800)
    gb = _sc_gather(b_t.reshape(T // 2 * BN, C), dst_all, P=800)

    consts_e = (jnp.zeros((8, C), f32)
                .at[0].set(w_d).at[1].set(b_e1).at[2].set(b_e2)
                .at[3].set(W_g[:, 0]).at[4, 0].set(b_g[0]))
    m_edges = _edge_mlp(ga, gb, d2, W_e2, consts_e, NEB=2000)

    BNP = ((BN // _NS + 7) // 8 * 8) * _NS
    zeros_nc = jnp.zeros((BNP, C), f32)
    agg = _sc_scatter(m_edges.reshape(T, E, C), dst, zeros_nc, P=200)

    consts_n = jnp.zeros((8, C), f32).at[0].set(b_n1).at[1].set(b_n2)
    h_out = _node_mlp(h2.reshape(T * BN, C), agg,
                      n1a, n1b, W_n2, consts_n, NB=2000)
    return h_out.reshape(T, BN, C), vel_new
